# baseline (speedup 1.0000x reference)
import functools

import jax
import jax.numpy as jnp
from jax import lax
from jax.experimental import pallas as pl
from jax.experimental.pallas import tpu as pltpu
from jax.experimental.pallas import tpu_sc as plsc

N = 10000
D = 128
E = 320000
T = 3
INV_DEG = 1.0 / 32.0

NC = 2
NS = 16
NW = NC * NS
CHUNK = 128
CPW = 83
EP = NW * CPW * CHUNK
ACC_ROWS = 10240
RPT = ACC_ROWS // NS
TAIL = N - (NS - 1) * RPT
DUMP = N

BR = 2000


def _seg_sum_body(m_hbm, src_hbm, dst_hbm, zero_hbm, out_hbm,
                  srcbuf, dstbuf, rows, gsem, acc):
    c = lax.axis_index("c")
    s = lax.axis_index("s")
    wid = c * NS + s

    pltpu.sync_copy(zero_hbm, acc.at[pl.ds(s * RPT, RPT)])
    plsc.subcore_barrier()

    base0 = wid * (CPW * CHUNK)

    def body(i, carry):
        off = base0 + i * CHUNK
        pltpu.sync_copy(src_hbm.at[pl.ds(off, CHUNK)], srcbuf)
        pltpu.sync_copy(dst_hbm.at[pl.ds(off, CHUNK)], dstbuf)
        pltpu.async_copy(m_hbm.at[srcbuf], rows, gsem).wait()
        pltpu.sync_copy(rows, acc.at[dstbuf], add=True)
        return carry

    lax.fori_loop(0, CPW, body, 0)
    plsc.subcore_barrier()

    @pl.when(s < NS - 1)
    def _():
        pltpu.sync_copy(acc.at[pl.ds(s * RPT, RPT)],
                        out_hbm.at[c, pl.ds(s * RPT, RPT)])

    @pl.when(s == NS - 1)
    def _():
        pltpu.sync_copy(acc.at[pl.ds((NS - 1) * RPT, TAIL)],
                        out_hbm.at[c, pl.ds((NS - 1) * RPT, TAIL)])


_seg_sum = pl.kernel(
    _seg_sum_body,
    out_type=jax.ShapeDtypeStruct((NC, N, D), jnp.float32),
    mesh=plsc.VectorSubcoreMesh(core_axis_name="c", subcore_axis_name="s"),
    scratch_types=[
        pltpu.VMEM((CHUNK,), jnp.int32),
        pltpu.VMEM((CHUNK,), jnp.int32),
        pltpu.VMEM((CHUNK, D), jnp.float32),
        pltpu.SemaphoreType.DMA,
        pltpu.VMEM_SHARED((ACC_ROWS, D), jnp.float32),
    ],
)


def _precompute_body(h_ref, wh_ref, s_ref, base_ref, m0_ref):
    cvec = jnp.mean(s_ref[...], axis=0, keepdims=True)
    b = lax.dot_general(h_ref[...], wh_ref[...], (((1,), (1,)), ((), ())),
                        preferred_element_type=jnp.float32) + cvec
    base_ref[...] = b
    m0_ref[...] = jnp.tanh(b)


def _precompute(h, Wh, s):
    return pl.pallas_call(
        _precompute_body,
        grid=(N // BR,),
        in_specs=[
            pl.BlockSpec((BR, D), lambda i: (i, 0)),
            pl.BlockSpec((D, D), lambda i: (0, 0)),
            pl.BlockSpec((20, D), lambda i: (0, 0)),
        ],
        out_specs=[
            pl.BlockSpec((BR, D), lambda i: (i, 0)),
            pl.BlockSpec((BR, D), lambda i: (i, 0)),
        ],
        out_shape=[
            jax.ShapeDtypeStruct((N, D), jnp.float32),
            jax.ShapeDtypeStruct((N, D), jnp.float32),
        ],
    )(h, Wh, s)


def _update_body(base_ref, p_ref, m_ref):
    ctx = (p_ref[0] + p_ref[1]) * INV_DEG
    m_ref[...] = jnp.tanh(base_ref[...] + ctx)


def _update(base, p):
    return pl.pallas_call(
        _update_body,
        grid=(N // BR,),
        in_specs=[
            pl.BlockSpec((BR, D), lambda i: (i, 0)),
            pl.BlockSpec((NC, BR, D), lambda i: (0, i, 0)),
        ],
        out_specs=pl.BlockSpec((BR, D), lambda i: (i, 0)),
        out_shape=jax.ShapeDtypeStruct((N, D), jnp.float32),
    )(base, p)


def _final_body(h_ref, p_ref, wa_ref, wb_ref, out_ref):
    ctx = (p_ref[0] + p_ref[1]) * INV_DEG
    out_ref[...] = (
        lax.dot_general(h_ref[...], wa_ref[...], (((1,), (1,)), ((), ())),
                        preferred_element_type=jnp.float32)
        + lax.dot_general(ctx, wb_ref[...], (((1,), (1,)), ((), ())),
                          preferred_element_type=jnp.float32)
    )


def _final(h, p, Wa, Wb):
    return pl.pallas_call(
        _final_body,
        grid=(N // BR,),
        in_specs=[
            pl.BlockSpec((BR, D), lambda i: (i, 0)),
            pl.BlockSpec((NC, BR, D), lambda i: (0, i, 0)),
            pl.BlockSpec((D, D), lambda i: (0, 0)),
            pl.BlockSpec((D, D), lambda i: (0, 0)),
        ],
        out_specs=pl.BlockSpec((BR, D), lambda i: (i, 0)),
        out_shape=jax.ShapeDtypeStruct((N, D), jnp.float32),
    )(h, p, Wa, Wb)


def kernel(h, edge_index, s, Wh, W12):
    src = edge_index[0].astype(jnp.int32)
    dst = edge_index[1].astype(jnp.int32)
    pad = EP - E
    src = jnp.concatenate([src, jnp.zeros((pad,), jnp.int32)])
    dst = jnp.concatenate(
        [dst, DUMP + (jnp.arange(pad, dtype=jnp.int32) % (ACC_ROWS - N))])
    zero = jnp.zeros((RPT, D), jnp.float32)
    Wa = W12[:, :D]
    Wb = W12[:, D:]

    base, m = _precompute(h, Wh, s)
    p = None
    for t in range(T):
        p = _seg_sum(m, src, dst, zero)
        if t < T - 1:
            m = _update(base, p)
    return _final(h, p, Wa, Wb)


# speedup vs baseline: 3.2291x; 3.2291x over previous
import functools

import jax
import jax.numpy as jnp
from jax import lax
from jax.experimental import pallas as pl
from jax.experimental.pallas import tpu as pltpu
from jax.experimental.pallas import tpu_sc as plsc

N = 10000
D = 128
E = 320000
T = 3
INV_DEG = 1.0 / 32.0

NC = 2
NS = 16
NW = NC * NS
CHUNK = 128
CPW = 80
EP = NW * CPW * CHUNK
ACC_ROWS = 10240
RPT = ACC_ROWS // NS
TAIL = N - (NS - 1) * RPT
DUMP = N

BR = 2000


def _seg_sum_body(m_hbm, src_hbm, dst_hbm, zero_hbm, out_hbm,
                  srcbuf, dstbuf, rows, gsem, acc):
    c = lax.axis_index("c")
    s = lax.axis_index("s")
    wid = c * NS + s

    pltpu.sync_copy(zero_hbm, acc.at[pl.ds(s * RPT, RPT)])
    plsc.subcore_barrier()

    base0 = wid * (CPW * CHUNK)

    def body(i, carry):
        off = base0 + i * CHUNK
        pltpu.sync_copy(src_hbm.at[pl.ds(off, CHUNK)], srcbuf)
        pltpu.sync_copy(dst_hbm.at[pl.ds(off, CHUNK)], dstbuf)
        pltpu.async_copy(m_hbm.at[srcbuf], rows, gsem).wait()
        pltpu.sync_copy(rows, acc.at[dstbuf], add=True)
        return carry

    lax.fori_loop(0, CPW, body, 0)
    plsc.subcore_barrier()

    @pl.when(s < NS - 1)
    def _():
        pltpu.sync_copy(acc.at[pl.ds(s * RPT, RPT)],
                        out_hbm.at[c, pl.ds(s * RPT, RPT)])

    @pl.when(s == NS - 1)
    def _():
        pltpu.sync_copy(acc.at[pl.ds((NS - 1) * RPT, TAIL)],
                        out_hbm.at[c, pl.ds((NS - 1) * RPT, TAIL)])


_seg_sum = pl.kernel(
    _seg_sum_body,
    out_type=jax.ShapeDtypeStruct((NC, N, D), jnp.float32),
    mesh=plsc.VectorSubcoreMesh(core_axis_name="c", subcore_axis_name="s"),
    scratch_types=[
        pltpu.VMEM((CHUNK,), jnp.int32),
        pltpu.VMEM((CHUNK,), jnp.int32),
        pltpu.VMEM((CHUNK, D), jnp.float32),
        pltpu.SemaphoreType.DMA,
        pltpu.VMEM_SHARED((ACC_ROWS, D), jnp.float32),
    ],
)


def _precompute_body(h_ref, wh_ref, s_ref, base_ref, m0_ref):
    cvec = jnp.mean(s_ref[...], axis=0, keepdims=True)
    b = lax.dot_general(h_ref[...], wh_ref[...], (((1,), (1,)), ((), ())),
                        preferred_element_type=jnp.float32) + cvec
    base_ref[...] = b
    m0_ref[...] = jnp.tanh(b)


def _precompute(h, Wh, s):
    return pl.pallas_call(
        _precompute_body,
        grid=(N // BR,),
        in_specs=[
            pl.BlockSpec((BR, D), lambda i: (i, 0)),
            pl.BlockSpec((D, D), lambda i: (0, 0)),
            pl.BlockSpec((20, D), lambda i: (0, 0)),
        ],
        out_specs=[
            pl.BlockSpec((BR, D), lambda i: (i, 0)),
            pl.BlockSpec((BR, D), lambda i: (i, 0)),
        ],
        out_shape=[
            jax.ShapeDtypeStruct((N, D), jnp.float32),
            jax.ShapeDtypeStruct((N, D), jnp.float32),
        ],
    )(h, Wh, s)


def _update_body(base_ref, p_ref, m_ref):
    ctx = (p_ref[0] + p_ref[1]) * INV_DEG
    m_ref[...] = jnp.tanh(base_ref[...] + ctx)


def _update(base, p):
    return pl.pallas_call(
        _update_body,
        grid=(N // BR,),
        in_specs=[
            pl.BlockSpec((BR, D), lambda i: (i, 0)),
            pl.BlockSpec((NC, BR, D), lambda i: (0, i, 0)),
        ],
        out_specs=pl.BlockSpec((BR, D), lambda i: (i, 0)),
        out_shape=jax.ShapeDtypeStruct((N, D), jnp.float32),
    )(base, p)


def _final_body(h_ref, p_ref, wa_ref, wb_ref, out_ref):
    ctx = (p_ref[0] + p_ref[1]) * INV_DEG
    out_ref[...] = (
        lax.dot_general(h_ref[...], wa_ref[...], (((1,), (1,)), ((), ())),
                        preferred_element_type=jnp.float32)
        + lax.dot_general(ctx, wb_ref[...], (((1,), (1,)), ((), ())),
                          preferred_element_type=jnp.float32)
    )


def _final(h, p, Wa, Wb):
    return pl.pallas_call(
        _final_body,
        grid=(N // BR,),
        in_specs=[
            pl.BlockSpec((BR, D), lambda i: (i, 0)),
            pl.BlockSpec((NC, BR, D), lambda i: (0, i, 0)),
            pl.BlockSpec((D, D), lambda i: (0, 0)),
            pl.BlockSpec((D, D), lambda i: (0, 0)),
        ],
        out_specs=pl.BlockSpec((BR, D), lambda i: (i, 0)),
        out_shape=jax.ShapeDtypeStruct((N, D), jnp.float32),
    )(h, p, Wa, Wb)


def kernel(h, edge_index, s, Wh, W12):
    src = edge_index[0].astype(jnp.int32)
    dst = edge_index[1].astype(jnp.int32)
    pad = EP - E
    src = jnp.concatenate([src, src[:pad]])
    dst = jnp.concatenate(
        [dst, DUMP + (jnp.arange(pad, dtype=jnp.int32) % (ACC_ROWS - N))])
    zero = jnp.zeros((RPT, D), jnp.float32)
    Wa = W12[:, :D]
    Wb = W12[:, D:]

    base, m = _precompute(h, Wh, s)
    p = None
    for t in range(T):
        p = _seg_sum(m, src, dst, zero)
        if t < T - 1:
            m = _update(base, p)
    return _final(h, p, Wa, Wb)


# speedup vs baseline: 4.3525x; 1.3479x over previous
import functools

import jax
import jax.numpy as jnp
from jax import lax
from jax.experimental import pallas as pl
from jax.experimental.pallas import tpu as pltpu
from jax.experimental.pallas import tpu_sc as plsc

N = 10000
D = 128
E = 320000
T = 3
INV_DEG = 1.0 / 32.0

NC = 2
NS = 16
NW = NC * NS
CHUNK = 128
CPW = 80
EP = NW * CPW * CHUNK
ACC_ROWS = 10240
RPT = ACC_ROWS // NS
TAIL = N - (NS - 1) * RPT
DUMP = N

BR = 2000


def _seg_sum_body(m_hbm, idx_hbm, zero_hbm, out_hbm,
                  islab, rows, gsem, acc):
    c = lax.axis_index("c")
    s = lax.axis_index("s")
    wid = c * NS + s

    pltpu.sync_copy(idx_hbm.at[wid], islab)
    pltpu.sync_copy(zero_hbm, acc.at[pl.ds(s * RPT, RPT)])
    plsc.subcore_barrier()

    def body(i, carry):
        pltpu.async_copy(m_hbm.at[islab.at[i, 0]], rows, gsem).wait()
        pltpu.sync_copy(rows, acc.at[islab.at[i, 1]], add=True)
        return carry

    lax.fori_loop(0, CPW, body, 0)
    plsc.subcore_barrier()

    @pl.when(s < NS - 1)
    def _():
        pltpu.sync_copy(acc.at[pl.ds(s * RPT, RPT)],
                        out_hbm.at[c, pl.ds(s * RPT, RPT)])

    @pl.when(s == NS - 1)
    def _():
        pltpu.sync_copy(acc.at[pl.ds((NS - 1) * RPT, TAIL)],
                        out_hbm.at[c, pl.ds((NS - 1) * RPT, TAIL)])


_seg_sum = pl.kernel(
    _seg_sum_body,
    out_type=jax.ShapeDtypeStruct((NC, N, D), jnp.float32),
    mesh=plsc.VectorSubcoreMesh(core_axis_name="c", subcore_axis_name="s"),
    scratch_types=[
        pltpu.VMEM((CPW, 2, CHUNK), jnp.int32),
        pltpu.VMEM((CHUNK, D), jnp.float32),
        pltpu.SemaphoreType.DMA,
        pltpu.VMEM_SHARED((ACC_ROWS, D), jnp.float32),
    ],
)


def _precompute_body(h_ref, wh_ref, s_ref, base_ref, m0_ref):
    cvec = jnp.mean(s_ref[...], axis=0, keepdims=True)
    b = lax.dot_general(h_ref[...], wh_ref[...], (((1,), (1,)), ((), ())),
                        preferred_element_type=jnp.float32) + cvec
    base_ref[...] = b
    m0_ref[...] = jnp.tanh(b)


def _precompute(h, Wh, s):
    return pl.pallas_call(
        _precompute_body,
        grid=(N // BR,),
        in_specs=[
            pl.BlockSpec((BR, D), lambda i: (i, 0)),
            pl.BlockSpec((D, D), lambda i: (0, 0)),
            pl.BlockSpec((20, D), lambda i: (0, 0)),
        ],
        out_specs=[
            pl.BlockSpec((BR, D), lambda i: (i, 0)),
            pl.BlockSpec((BR, D), lambda i: (i, 0)),
        ],
        out_shape=[
            jax.ShapeDtypeStruct((N, D), jnp.float32),
            jax.ShapeDtypeStruct((N, D), jnp.float32),
        ],
    )(h, Wh, s)


def _update_body(base_ref, p_ref, m_ref):
    ctx = (p_ref[0] + p_ref[1]) * INV_DEG
    m_ref[...] = jnp.tanh(base_ref[...] + ctx)


def _update(base, p):
    return pl.pallas_call(
        _update_body,
        grid=(N // BR,),
        in_specs=[
            pl.BlockSpec((BR, D), lambda i: (i, 0)),
            pl.BlockSpec((NC, BR, D), lambda i: (0, i, 0)),
        ],
        out_specs=pl.BlockSpec((BR, D), lambda i: (i, 0)),
        out_shape=jax.ShapeDtypeStruct((N, D), jnp.float32),
    )(base, p)


def _final_body(h_ref, p_ref, wa_ref, wb_ref, out_ref):
    ctx = (p_ref[0] + p_ref[1]) * INV_DEG
    out_ref[...] = (
        lax.dot_general(h_ref[...], wa_ref[...], (((1,), (1,)), ((), ())),
                        preferred_element_type=jnp.float32)
        + lax.dot_general(ctx, wb_ref[...], (((1,), (1,)), ((), ())),
                          preferred_element_type=jnp.float32)
    )


def _final(h, p, Wa, Wb):
    return pl.pallas_call(
        _final_body,
        grid=(N // BR,),
        in_specs=[
            pl.BlockSpec((BR, D), lambda i: (i, 0)),
            pl.BlockSpec((NC, BR, D), lambda i: (0, i, 0)),
            pl.BlockSpec((D, D), lambda i: (0, 0)),
            pl.BlockSpec((D, D), lambda i: (0, 0)),
        ],
        out_specs=pl.BlockSpec((BR, D), lambda i: (i, 0)),
        out_shape=jax.ShapeDtypeStruct((N, D), jnp.float32),
    )(h, p, Wa, Wb)


def kernel(h, edge_index, s, Wh, W12):
    src = edge_index[0].astype(jnp.int32)
    dst = edge_index[1].astype(jnp.int32)
    pad = EP - E
    src = jnp.concatenate([src, src[:pad]])
    dst = jnp.concatenate(
        [dst, DUMP + (jnp.arange(pad, dtype=jnp.int32) % (ACC_ROWS - N))])
    idx = jnp.stack([src.reshape(NW, CPW, CHUNK),
                     dst.reshape(NW, CPW, CHUNK)], axis=2)
    zero = jnp.zeros((RPT, D), jnp.float32)
    Wa = W12[:, :D]
    Wb = W12[:, D:]

    base, m = _precompute(h, Wh, s)
    p = None
    for t in range(T):
        p = _seg_sum(m, idx, zero)
        if t < T - 1:
            m = _update(base, p)
    return _final(h, p, Wa, Wb)


# speedup vs baseline: 6.6270x; 1.5226x over previous
import functools

import jax
import jax.numpy as jnp
from jax import lax
from jax.experimental import pallas as pl
from jax.experimental.pallas import tpu as pltpu
from jax.experimental.pallas import tpu_sc as plsc

N = 10000
D = 128
E = 320000
T = 3
INV_DEG = 1.0 / 32.0

NC = 2
NS = 16
NW = NC * NS
CHUNK = 128
CPW = 80
EP = NW * CPW * CHUNK
ACC_ROWS = 10240
RPT = ACC_ROWS // NS
TAIL = N - (NS - 1) * RPT
DUMP = N

BR = 2000


def _seg_sum_body(m_hbm, srcs_hbm, dst_hbm, zero_hbm, out_hbm,
                  sslab, dbuf0, dbuf1, rows0, rows1,
                  gsem0, gsem1, dsem0, dsem1, acc):
    c = lax.axis_index("c")
    s = lax.axis_index("s")
    wid = c * NS + s
    k0 = wid * CPW

    pltpu.sync_copy(srcs_hbm.at[wid], sslab)
    pltpu.sync_copy(zero_hbm, acc.at[pl.ds(s * RPT, RPT)])
    plsc.subcore_barrier()

    pltpu.async_copy(dst_hbm.at[k0], dbuf0, dsem0)
    pltpu.async_copy(dst_hbm.at[k0 + 1], dbuf1, dsem1)
    pltpu.async_copy(m_hbm.at[sslab.at[0]], rows0, gsem0)
    pltpu.async_copy(m_hbm.at[sslab.at[1]], rows1, gsem1)

    def body(j, carry):
        i0 = 2 * j
        pltpu.make_async_copy(m_hbm.at[sslab.at[i0]], rows0, gsem0).wait()
        pltpu.make_async_copy(dst_hbm.at[k0 + i0], dbuf0, dsem0).wait()
        pltpu.sync_copy(rows0, acc.at[dbuf0], add=True)
        pltpu.async_copy(dst_hbm.at[k0 + i0 + 2], dbuf0, dsem0)
        pltpu.async_copy(m_hbm.at[sslab.at[i0 + 2]], rows0, gsem0)

        pltpu.make_async_copy(m_hbm.at[sslab.at[i0 + 1]], rows1, gsem1).wait()
        pltpu.make_async_copy(dst_hbm.at[k0 + i0 + 1], dbuf1, dsem1).wait()
        pltpu.sync_copy(rows1, acc.at[dbuf1], add=True)
        pltpu.async_copy(dst_hbm.at[k0 + i0 + 3], dbuf1, dsem1)
        pltpu.async_copy(m_hbm.at[sslab.at[i0 + 3]], rows1, gsem1)
        return carry

    lax.fori_loop(0, CPW // 2, body, 0)
    pltpu.make_async_copy(m_hbm.at[sslab.at[CPW]], rows0, gsem0).wait()
    pltpu.make_async_copy(m_hbm.at[sslab.at[CPW + 1]], rows1, gsem1).wait()
    pltpu.make_async_copy(dst_hbm.at[k0 + CPW], dbuf0, dsem0).wait()
    pltpu.make_async_copy(dst_hbm.at[k0 + CPW + 1], dbuf1, dsem1).wait()
    plsc.subcore_barrier()

    @pl.when(s < NS - 1)
    def _():
        pltpu.sync_copy(acc.at[pl.ds(s * RPT, RPT)],
                        out_hbm.at[c, pl.ds(s * RPT, RPT)])

    @pl.when(s == NS - 1)
    def _():
        pltpu.sync_copy(acc.at[pl.ds((NS - 1) * RPT, TAIL)],
                        out_hbm.at[c, pl.ds((NS - 1) * RPT, TAIL)])


_seg_sum = pl.kernel(
    _seg_sum_body,
    out_type=jax.ShapeDtypeStruct((NC, N, D), jnp.float32),
    mesh=plsc.VectorSubcoreMesh(core_axis_name="c", subcore_axis_name="s"),
    scratch_types=[
        pltpu.VMEM((CPW + 2, CHUNK), jnp.int32),
        pltpu.VMEM((CHUNK,), jnp.int32),
        pltpu.VMEM((CHUNK,), jnp.int32),
        pltpu.VMEM((CHUNK, D), jnp.float32),
        pltpu.VMEM((CHUNK, D), jnp.float32),
        pltpu.SemaphoreType.DMA,
        pltpu.SemaphoreType.DMA,
        pltpu.SemaphoreType.DMA,
        pltpu.SemaphoreType.DMA,
        pltpu.VMEM_SHARED((ACC_ROWS, D), jnp.float32),
    ],
)


def _precompute_body(h_ref, wh_ref, s_ref, base_ref, m0_ref):
    cvec = jnp.mean(s_ref[...], axis=0, keepdims=True)
    b = lax.dot_general(h_ref[...], wh_ref[...], (((1,), (1,)), ((), ())),
                        preferred_element_type=jnp.float32) + cvec
    base_ref[...] = b
    m0_ref[...] = jnp.tanh(b)


def _precompute(h, Wh, s):
    return pl.pallas_call(
        _precompute_body,
        grid=(N // BR,),
        in_specs=[
            pl.BlockSpec((BR, D), lambda i: (i, 0)),
            pl.BlockSpec((D, D), lambda i: (0, 0)),
            pl.BlockSpec((20, D), lambda i: (0, 0)),
        ],
        out_specs=[
            pl.BlockSpec((BR, D), lambda i: (i, 0)),
            pl.BlockSpec((BR, D), lambda i: (i, 0)),
        ],
        out_shape=[
            jax.ShapeDtypeStruct((N, D), jnp.float32),
            jax.ShapeDtypeStruct((N, D), jnp.float32),
        ],
    )(h, Wh, s)


def _update_body(base_ref, p_ref, m_ref):
    ctx = (p_ref[0] + p_ref[1]) * INV_DEG
    m_ref[...] = jnp.tanh(base_ref[...] + ctx)


def _update(base, p):
    return pl.pallas_call(
        _update_body,
        grid=(N // BR,),
        in_specs=[
            pl.BlockSpec((BR, D), lambda i: (i, 0)),
            pl.BlockSpec((NC, BR, D), lambda i: (0, i, 0)),
        ],
        out_specs=pl.BlockSpec((BR, D), lambda i: (i, 0)),
        out_shape=jax.ShapeDtypeStruct((N, D), jnp.float32),
    )(base, p)


def _final_body(h_ref, p_ref, wa_ref, wb_ref, out_ref):
    ctx = (p_ref[0] + p_ref[1]) * INV_DEG
    out_ref[...] = (
        lax.dot_general(h_ref[...], wa_ref[...], (((1,), (1,)), ((), ())),
                        preferred_element_type=jnp.float32)
        + lax.dot_general(ctx, wb_ref[...], (((1,), (1,)), ((), ())),
                          preferred_element_type=jnp.float32)
    )


def _final(h, p, Wa, Wb):
    return pl.pallas_call(
        _final_body,
        grid=(N // BR,),
        in_specs=[
            pl.BlockSpec((BR, D), lambda i: (i, 0)),
            pl.BlockSpec((NC, BR, D), lambda i: (0, i, 0)),
            pl.BlockSpec((D, D), lambda i: (0, 0)),
            pl.BlockSpec((D, D), lambda i: (0, 0)),
        ],
        out_specs=pl.BlockSpec((BR, D), lambda i: (i, 0)),
        out_shape=jax.ShapeDtypeStruct((N, D), jnp.float32),
    )(h, p, Wa, Wb)


def kernel(h, edge_index, s, Wh, W12):
    src = edge_index[0].astype(jnp.int32)
    dst = edge_index[1].astype(jnp.int32)
    pad = EP - E + 2 * CHUNK
    src = jnp.concatenate([src, src[:pad]])
    dst = jnp.concatenate(
        [dst, DUMP + (jnp.arange(pad, dtype=jnp.int32) % (ACC_ROWS - N))])
    rowidx = (jnp.arange(NW, dtype=jnp.int32)[:, None] * CPW
              + jnp.arange(CPW + 2, dtype=jnp.int32)[None, :])
    srcs = src.reshape(NW * CPW + 2, CHUNK)[rowidx]
    dst = dst.reshape(NW * CPW + 2, CHUNK)
    zero = jnp.zeros((RPT, D), jnp.float32)
    Wa = W12[:, :D]
    Wb = W12[:, D:]

    base, m = _precompute(h, Wh, s)
    p = None
    for t in range(T):
        p = _seg_sum(m, srcs, dst, zero)
        if t < T - 1:
            m = _update(base, p)
    return _final(h, p, Wa, Wb)
